# baseline (speedup 1.0000x reference)
import functools

import jax
import jax.numpy as jnp
from jax import lax
from jax.experimental import pallas as pl
from jax.experimental.pallas import tpu as pltpu
from jax.experimental.pallas import tpu_sc as plsc

B = 16384
F_SPARSE = 26
F_DENSE = 13
VOCAB = 1000000
VPAD = 1000448
L = 16
NC = 2
NS = 16
NW = NC * NS
ROWS = B // NW
NIDX = ROWS * F_SPARSE
NCH = ROWS // L


FLAT = F_SPARSE * VPAD
CCH = 2048
MAINCOLS = 997376
NTASK = 512
TASKS_PER_W = NTASK // NW
LASTCOL = MAINCOLS - CCH
TAILW = 3072
TAILN = 2688


def _sc_copy_body(table_hbm, tail_hbm, flat_hbm, buf0, buf1, sem0, sem1, sem2):
    wid = lax.axis_index("s") * NC + lax.axis_index("c")
    ft = wid % F_SPARSE
    tail = pltpu.async_copy(
        tail_hbm.at[pl.ds(pl.multiple_of(ft * TAILW, 1024), TAILN)],
        flat_hbm.at[pl.ds(pl.multiple_of(ft * VPAD + MAINCOLS, 128), TAILN)],
        sem2,
    )
    bufs = (buf0, buf1)
    sems = (sem0, sem1)

    def task_col(k):
        return pl.multiple_of(
            jnp.minimum((wid + k * NW) * CCH, LASTCOL) * 1, 128
        )

    def fire_outs(b, col):
        def fb(f, _):
            pltpu.async_copy(
                bufs[b].at[f, pl.ds(0, CCH)],
                flat_hbm.at[pl.ds(pl.multiple_of(f * VPAD + col, 128), CCH)],
                sems[b],
            )
            return ()
        lax.fori_loop(0, F_SPARSE, fb, ())

    def drain(b):
        def db(f, _):
            pltpu.make_async_copy(
                bufs[b].at[0, pl.ds(0, CCH)],
                flat_hbm.at[pl.ds(0, CCH)],
                sems[b],
            ).wait()
            return ()
        lax.fori_loop(0, F_SPARSE, db, ())

    for b in (0, 1):
        col = task_col(b)
        pltpu.sync_copy(table_hbm.at[:, pl.ds(col, CCH)], bufs[b])
        fire_outs(b, col)

    def pair(m, _):
        for b in (0, 1):
            col = task_col(2 * m + b)
            drain(b)
            pltpu.sync_copy(table_hbm.at[:, pl.ds(col, CCH)], bufs[b])
            fire_outs(b, col)
        return ()

    lax.fori_loop(1, TASKS_PER_W // 2, pair, ())
    drain(0)
    drain(1)
    tail.wait()


def _sc_body(xs_hbm, xd_hbm, table_hbm, wrep_hbm, out_hbm,
             xs_v, xd_v, wrep_v, idx_v, vals_v, acc_v, sem):
    wid = lax.axis_index("s") * NC + lax.axis_index("c")
    base = wid * ROWS

    pltpu.sync_copy(xs_hbm.at[:, pl.ds(base, ROWS)], xs_v)
    pltpu.sync_copy(xd_hbm.at[:, pl.ds(base, ROWS)], xd_v)
    pltpu.sync_copy(wrep_hbm, wrep_v)

    for f in range(F_SPARSE):
        off = f * VPAD
        for j in range(NCH):
            sl = pl.ds(j * L, L)
            idx_v[pl.ds(f * ROWS + j * L, L)] = xs_v[f, sl] + off

    pltpu.async_copy(table_hbm.at[idx_v], vals_v, sem).wait()

    wk = [wrep_v[pl.ds(k * L, L)] for k in range(F_DENSE)]
    for j in range(NCH):
        sl = pl.ds(j * L, L)
        acc = vals_v[pl.ds(j * L, L)]
        for f in range(1, F_SPARSE):
            acc = acc + vals_v[pl.ds(f * ROWS + j * L, L)]
        for k in range(F_DENSE):
            acc = acc + xd_v[k, sl] * wk[k]
        acc_v[sl] = 1.0 / (1.0 + jnp.exp(-acc))

    pltpu.sync_copy(acc_v, out_hbm.at[pl.ds(base, ROWS)])


@jax.jit
def kernel(X_sparse, X_dense, lin_table, W):
    xs_t = X_sparse.T
    xd_t = X_dense.T
    table2d = lin_table.reshape(F_SPARSE, VOCAB)
    wrep = jnp.repeat(W.reshape(F_DENSE), L)

    mesh = plsc.VectorSubcoreMesh(core_axis_name="c", subcore_axis_name="s")
    copy_run = pl.kernel(
        _sc_copy_body,
        out_type=jax.ShapeDtypeStruct((FLAT,), jnp.float32),
        mesh=mesh,
        scratch_types=[
            pltpu.VMEM((F_SPARSE, CCH), jnp.float32),
            pltpu.VMEM((F_SPARSE, CCH), jnp.float32),
            pltpu.SemaphoreType.DMA,
            pltpu.SemaphoreType.DMA,
            pltpu.SemaphoreType.DMA,
        ],
    )
    tail3 = jnp.pad(lin_table[:, MAINCOLS:, :], ((0, 0), (0, TAILW - (VOCAB - MAINCOLS)), (0, 0)))
    tail_flat = tail3.reshape(-1)
    table = copy_run(table2d, tail_flat)
    run = pl.kernel(
        _sc_body,
        out_type=jax.ShapeDtypeStruct((B,), jnp.float32),
        mesh=mesh,
        scratch_types=[
            pltpu.VMEM((F_SPARSE, ROWS), jnp.int32),
            pltpu.VMEM((F_DENSE, ROWS), jnp.float32),
            pltpu.VMEM((F_DENSE * L,), jnp.float32),
            pltpu.VMEM((NIDX,), jnp.int32),
            pltpu.VMEM((NIDX,), jnp.float32),
            pltpu.VMEM((ROWS,), jnp.float32),
            pltpu.SemaphoreType.DMA,
        ],
    )
    out = run(xs_t, xd_t, table, wrep)
    return out.reshape(B, 1)


# speedup vs baseline: 2.2985x; 2.2985x over previous
import functools

import jax
import jax.numpy as jnp
from jax import lax
from jax.experimental import pallas as pl
from jax.experimental.pallas import tpu as pltpu
from jax.experimental.pallas import tpu_sc as plsc

B = 16384
F_SPARSE = 26
F_DENSE = 13
VOCAB = 1000000
VPAD = 1000448
L = 16
NC = 2
NS = 16
NW = NC * NS
ROWS = B // NW
NIDX = ROWS * F_SPARSE
NCH = ROWS // L


def _sc_body(xs_hbm, xd_hbm, table_hbm, wrep_hbm, out_hbm,
             xs_v, xd_v, wrep_v, idx_v, vals_v, acc_v, sem):
    wid = lax.axis_index("s") * NC + lax.axis_index("c")
    base = wid * ROWS

    pltpu.sync_copy(xs_hbm.at[:, pl.ds(base, ROWS)], xs_v)
    pltpu.sync_copy(xd_hbm.at[:, pl.ds(base, ROWS)], xd_v)
    pltpu.sync_copy(wrep_hbm, wrep_v)

    FH = F_SPARSE // 2
    def build(f_lo, f_hi):
        for f in range(f_lo, f_hi):
            off = f * VPAD
            for j in range(NCH):
                sl = pl.ds(j * L, L)
                idx_v[pl.ds(f * ROWS + j * L, L)] = xs_v[f, sl] + off

    build(0, FH)
    g0 = pltpu.async_copy(
        table_hbm.at[idx_v.at[pl.ds(0, FH * ROWS)]],
        vals_v.at[pl.ds(0, FH * ROWS)], sem)
    build(FH, F_SPARSE)
    g1 = pltpu.async_copy(
        table_hbm.at[idx_v.at[pl.ds(FH * ROWS, (F_SPARSE - FH) * ROWS)]],
        vals_v.at[pl.ds(FH * ROWS, (F_SPARSE - FH) * ROWS)], sem)
    g0.wait()
    g1.wait()

    wk = [wrep_v[pl.ds(k * L, L)] for k in range(F_DENSE)]
    for j in range(NCH):
        sl = pl.ds(j * L, L)
        acc = vals_v[pl.ds(j * L, L)]
        for f in range(1, F_SPARSE):
            acc = acc + vals_v[pl.ds(f * ROWS + j * L, L)]
        for k in range(F_DENSE):
            acc = acc + xd_v[k, sl] * wk[k]
        acc_v[sl] = 1.0 / (1.0 + jnp.exp(-acc))

    pltpu.sync_copy(acc_v, out_hbm.at[pl.ds(base, ROWS)])


@jax.jit
def kernel(X_sparse, X_dense, lin_table, W):
    xs_t = X_sparse.T
    xd_t = X_dense.T
    table = jnp.pad(lin_table, ((0, 0), (0, VPAD - VOCAB), (0, 0))).reshape(-1)
    wrep = jnp.repeat(W.reshape(F_DENSE), L)

    mesh = plsc.VectorSubcoreMesh(core_axis_name="c", subcore_axis_name="s")
    run = pl.kernel(
        _sc_body,
        out_type=jax.ShapeDtypeStruct((B,), jnp.float32),
        mesh=mesh,
        scratch_types=[
            pltpu.VMEM((F_SPARSE, ROWS), jnp.int32),
            pltpu.VMEM((F_DENSE, ROWS), jnp.float32),
            pltpu.VMEM((F_DENSE * L,), jnp.float32),
            pltpu.VMEM((NIDX,), jnp.int32),
            pltpu.VMEM((NIDX,), jnp.float32),
            pltpu.VMEM((ROWS,), jnp.float32),
            pltpu.SemaphoreType.DMA,
        ],
    )
    out = run(xs_t, xd_t, table, wrep)
    return out.reshape(B, 1)
